# Initial kernel scaffold; baseline (speedup 1.0000x reference)
import jax
import jax.numpy as jnp
from jax.experimental import pallas as pl


def kernel(input_ids, wte):
    raise NotImplementedError("write your pallas kernel here")


# speedup vs baseline: 6.3928x; 6.3928x over previous
import functools

import jax
import jax.numpy as jnp
from jax import lax
from jax.experimental import pallas as pl
from jax.experimental.pallas import tpu as pltpu
from jax.experimental.pallas import tpu_sc as plsc

D = 128
NUM_WORKERS = 32
CHUNK = 256


def _make_gather(B):
    b_per_w = B // NUM_WORKERS
    n_chunks = b_per_w // CHUNK
    mesh = plsc.VectorSubcoreMesh(core_axis_name="c", subcore_axis_name="s")

    @functools.partial(
        pl.kernel,
        mesh=mesh,
        out_type=jax.ShapeDtypeStruct((B, D), jnp.float32),
        scratch_types=[
            pltpu.VMEM((CHUNK,), jnp.int32),
            pltpu.VMEM((CHUNK, D), jnp.float32),
            pltpu.SemaphoreType.DMA,
        ],
    )
    def gather(idx_hbm, table_hbm, out_hbm, idx_v, rows_v, sem):
        wid = lax.axis_index("s") * 2 + lax.axis_index("c")
        base = wid * b_per_w

        def body(i, carry):
            off = base + i * CHUNK
            pltpu.sync_copy(idx_hbm.at[pl.ds(off, CHUNK)], idx_v)
            pltpu.async_copy(table_hbm.at[idx_v], rows_v, sem).wait()
            pltpu.sync_copy(rows_v, out_hbm.at[pl.ds(off, CHUNK)])
            return carry

        lax.fori_loop(0, n_chunks, body, 0)

    return gather


def kernel(input_ids, wte):
    B = input_ids.shape[0] * input_ids.shape[1]
    idx = input_ids.reshape(B)
    table = wte.astype(jnp.float32)
    out = _make_gather(B)(idx, table)
    return out.reshape(input_ids.shape + (D,))


# speedup vs baseline: 8.5569x; 1.3385x over previous
import functools

import jax
import jax.numpy as jnp
from jax import lax
from jax.experimental import pallas as pl
from jax.experimental.pallas import tpu as pltpu
from jax.experimental.pallas import tpu_sc as plsc

D = 128
NUM_WORKERS = 32
CHUNK = 128
NBUF = 4


def _make_gather(B):
    b_per_w = B // NUM_WORKERS
    n_chunks = b_per_w // CHUNK
    n_rounds = n_chunks // NBUF
    mesh = plsc.VectorSubcoreMesh(core_axis_name="c", subcore_axis_name="s")

    @functools.partial(
        pl.kernel,
        mesh=mesh,
        out_type=jax.ShapeDtypeStruct((B, D), jnp.float32),
        scratch_types=[
            pltpu.VMEM((NBUF, CHUNK), jnp.int32),
            pltpu.VMEM((NBUF, CHUNK, D), jnp.float32),
        ]
        + [pltpu.SemaphoreType.DMA] * (2 * NBUF),
    )
    def gather(idx_hbm, table_hbm, out_hbm, idx_v, rows_v, *sems):
        gsem = sems[:NBUF]
        ssem = sems[NBUF:]
        wid = lax.axis_index("s") * 2 + lax.axis_index("c")
        base = wid * b_per_w

        for b in range(NBUF):
            pltpu.sync_copy(idx_hbm.at[pl.ds(base + b * CHUNK, CHUNK)],
                            idx_v.at[b])
            pltpu.async_copy(table_hbm.at[idx_v.at[b]], rows_v.at[b], gsem[b])

        def body(r, carry):
            for b in range(NBUF):
                c_off = base + (r * NBUF + b) * CHUNK
                pltpu.make_async_copy(table_hbm.at[idx_v.at[b]],
                                      rows_v.at[b], gsem[b]).wait()
                pltpu.async_copy(rows_v.at[b],
                                 out_hbm.at[pl.ds(c_off, CHUNK)], ssem[b])

                @pl.when(r < n_rounds - 1)
                def _():
                    n_off = c_off + NBUF * CHUNK
                    pltpu.sync_copy(idx_hbm.at[pl.ds(n_off, CHUNK)],
                                    idx_v.at[b])
                    pltpu.make_async_copy(rows_v.at[b],
                                          out_hbm.at[pl.ds(c_off, CHUNK)],
                                          ssem[b]).wait()
                    pltpu.async_copy(table_hbm.at[idx_v.at[b]],
                                     rows_v.at[b], gsem[b])
            return carry

        lax.fori_loop(0, n_rounds, body, 0)

        last = base + (n_chunks - NBUF) * CHUNK
        for b in range(NBUF):
            pltpu.make_async_copy(
                rows_v.at[b],
                out_hbm.at[pl.ds(last + b * CHUNK, CHUNK)], ssem[b]).wait()

    return gather


def kernel(input_ids, wte):
    B = input_ids.shape[0] * input_ids.shape[1]
    idx = input_ids.reshape(B)
    table = wte.astype(jnp.float32)
    out = _make_gather(B)(idx, table)
    return out.reshape(input_ids.shape + (D,))
